# Initial kernel scaffold; baseline (speedup 1.0000x reference)
import jax
import jax.numpy as jnp
from jax.experimental import pallas as pl


def kernel(elements_info, elements_mask, W, b, tm_table):
    raise NotImplementedError("write your pallas kernel here")


# speedup vs baseline: 7.9322x; 7.9322x over previous
import jax
import jax.numpy as jnp
from jax.experimental import pallas as pl

B, L, F = 4096, 20, 7
O_LIN, O_EMB, O = 16, 8, 24
NTAB = 25
BM = 512


def _tc_body(x_ref, m_ref, a_ref, b_ref, r_ref, e_ref, k7_ref, k24_ref, o_ref):
    x = x_ref[...]
    m = m_ref[...]
    m7 = jnp.dot(m, k7_ref[...], preferred_element_type=jnp.float32)
    xm = x * m7
    lin = jax.nn.relu(
        jnp.dot(xm, a_ref[...], preferred_element_type=jnp.float32) + b_ref[...]
    )
    zrep = jnp.dot(xm, r_ref[...], preferred_element_type=jnp.float32)
    zi = zrep.astype(jnp.int32)
    k_iota = jax.lax.rem(jax.lax.broadcasted_iota(jnp.int32, zi.shape, 1), NTAB)
    mapped = jnp.where((zi >= 57) & (zi <= 80), zi - 56, 0)
    onehot = (mapped == k_iota).astype(jnp.float32)
    emb = jnp.dot(onehot, e_ref[...], preferred_element_type=jnp.float32)
    m24 = jnp.dot(m, k24_ref[...], preferred_element_type=jnp.float32)
    o_ref[...] = (lin + emb) * m24


def kernel(elements_info, elements_mask, W, b, tm_table):
    x2 = elements_info.reshape(B, L * F)
    eye = jnp.eye(L, dtype=jnp.float32)
    wblk = jnp.zeros((F, O), jnp.float32).at[:5, :O_LIN].set(W.T)
    A = jnp.kron(eye, wblk)
    bvec = jnp.tile(
        jnp.concatenate([b, jnp.zeros((O_EMB,), jnp.float32)]), (L,)
    )[None]
    rblk = jnp.zeros((F, NTAB), jnp.float32).at[5, :].set(1.0)
    R = jnp.kron(eye, rblk)
    eblk = jnp.zeros((NTAB, O), jnp.float32).at[:, O_LIN:].set(tm_table)
    E = jnp.kron(eye, eblk)
    K7 = jnp.kron(eye, jnp.ones((1, F), jnp.float32))
    K24 = jnp.kron(eye, jnp.ones((1, O), jnp.float32))
    out = pl.pallas_call(
        _tc_body,
        grid=(B // BM,),
        in_specs=[
            pl.BlockSpec((BM, L * F), lambda i: (i, 0)),
            pl.BlockSpec((BM, L), lambda i: (i, 0)),
            pl.BlockSpec((L * F, L * O), lambda i: (0, 0)),
            pl.BlockSpec((1, L * O), lambda i: (0, 0)),
            pl.BlockSpec((L * F, L * NTAB), lambda i: (0, 0)),
            pl.BlockSpec((L * NTAB, L * O), lambda i: (0, 0)),
            pl.BlockSpec((L, L * F), lambda i: (0, 0)),
            pl.BlockSpec((L, L * O), lambda i: (0, 0)),
        ],
        out_specs=pl.BlockSpec((BM, L * O), lambda i: (i, 0)),
        out_shape=jax.ShapeDtypeStruct((B, L * O), jnp.float32),
    )(x2, elements_mask, A, bvec, R, E, K7, K24)
    return out.reshape(B, L, O)
